# Initial kernel scaffold; baseline (speedup 1.0000x reference)
import jax
import jax.numpy as jnp
from jax.experimental import pallas as pl


def kernel(x, W1, W2, frozen1, frozen2, target1, target2):
    raise NotImplementedError("write your pallas kernel here")


# speedup vs baseline: 1.1003x; 1.1003x over previous
import jax
import jax.numpy as jnp
from jax.experimental import pallas as pl

_DFF = 4096
_DMODEL = 1024
_K = 1024
_MB = 512
_NBLK = _DFF // _MB


def _mags_body(w1_ref, w2_ref, out_ref):
    w1 = w1_ref[...]
    w2 = w2_ref[...]
    s1 = jnp.sum(w1 * w1, axis=1)
    s2 = jnp.sum(w2 * w2, axis=0)
    out_ref[...] = (jnp.sqrt(s1) * jnp.sqrt(s2)).reshape(1, 1, _MB)


def _mask_body(mags_ref, mask_ref):
    m = mags_ref[...]
    u = jax.lax.bitcast_convert_type(m, jnp.int32)
    k = jnp.int32(_K)

    def bs_body(_, carry):
        lo, hi = carry
        mid = lo + (hi - lo) // 2
        c = jnp.sum((u <= mid).astype(jnp.int32))
        take = c >= k
        return jnp.where(take, lo, mid + 1), jnp.where(take, mid, hi)

    _, p = jax.lax.fori_loop(
        0, 31, bs_body, (jnp.int32(0), jnp.int32(0x7F800000)))

    lt = u < p
    eq = u == p
    c_lt = jnp.sum(lt.astype(jnp.int32))
    need = k - c_lt

    e = eq.astype(jnp.int32)
    x = e
    s = 1
    while s < _MB:
        sh = jnp.concatenate([jnp.zeros((_NBLK, s), jnp.int32), x[:, :-s]],
                             axis=1)
        x = x + sh
        s *= 2
    row_tot = x[:, _MB - 1:_MB]
    y = row_tot
    s = 1
    while s < _NBLK:
        shy = jnp.concatenate([jnp.zeros((s, 1), jnp.int32), y[:-s, :]],
                              axis=0)
        y = y + shy
        s *= 2
    row_off = jnp.concatenate([jnp.zeros((1, 1), jnp.int32), y[:-1, :]],
                              axis=0)
    excl = (x - e) + row_off
    prune_eq = eq & (excl < need)
    keep = jnp.logical_not(jnp.logical_or(lt, prune_eq))
    mask_ref[...] = keep.astype(jnp.float32)


def _blend_rows_body(m_ref, w_ref, f_ref, out_ref):
    keep = m_ref[...] > 0.5
    out_ref[...] = jnp.where(keep, w_ref[...], f_ref[...])


def _blend_cols_body(m_ref, w_ref, f_ref, out_ref):
    keep = m_ref[...] > 0.5
    out_ref[...] = jnp.where(keep, w_ref[...], f_ref[...])


def _relu_body(x_ref, y_ref):
    y_ref[...] = jnp.maximum(x_ref[...], 0.0)


def kernel(x, W1, W2, frozen1, frozen2, target1, target2):
    del target1, target2

    mags = pl.pallas_call(
        _mags_body,
        grid=(_NBLK,),
        in_specs=[pl.BlockSpec((_MB, _DMODEL), lambda i: (i, 0)),
                  pl.BlockSpec((_DMODEL, _MB), lambda i: (0, i))],
        out_specs=pl.BlockSpec((1, 1, _MB), lambda i: (i, 0, 0)),
        out_shape=jax.ShapeDtypeStruct((_NBLK, 1, _MB), jnp.float32),
    )(W1, W2)

    mask2d = pl.pallas_call(
        _mask_body,
        out_shape=jax.ShapeDtypeStruct((_NBLK, _MB), jnp.float32),
    )(mags.reshape(_NBLK, _MB))

    mask = mask2d.reshape(_DFF)
    mask_col = mask2d.reshape(_DFF, 1)
    mask_row = mask2d.reshape(1, _DFF)

    bw = 512
    W1_new = pl.pallas_call(
        _blend_rows_body,
        grid=(_DFF // bw,),
        in_specs=[pl.BlockSpec((bw, 1), lambda i: (i, 0)),
                  pl.BlockSpec((bw, _DMODEL), lambda i: (i, 0)),
                  pl.BlockSpec((bw, _DMODEL), lambda i: (i, 0))],
        out_specs=pl.BlockSpec((bw, _DMODEL), lambda i: (i, 0)),
        out_shape=jax.ShapeDtypeStruct((_DFF, _DMODEL), jnp.float32),
    )(mask_col, W1, frozen1)

    W2_new = pl.pallas_call(
        _blend_cols_body,
        grid=(_DFF // bw,),
        in_specs=[pl.BlockSpec((1, bw), lambda i: (0, i)),
                  pl.BlockSpec((_DMODEL, bw), lambda i: (0, i)),
                  pl.BlockSpec((_DMODEL, bw), lambda i: (0, i))],
        out_specs=pl.BlockSpec((_DMODEL, bw), lambda i: (0, i)),
        out_shape=jax.ShapeDtypeStruct((_DMODEL, _DFF), jnp.float32),
    )(mask_row, W2, frozen2)

    x2 = x.reshape(-1, _DMODEL)
    xb = 1024
    y = pl.pallas_call(
        _relu_body,
        grid=(x2.shape[0] // xb,),
        in_specs=[pl.BlockSpec((xb, _DMODEL), lambda i: (i, 0))],
        out_specs=pl.BlockSpec((xb, _DMODEL), lambda i: (i, 0)),
        out_shape=jax.ShapeDtypeStruct(x2.shape, jnp.float32),
    )(x2)

    return y.reshape(x.shape), W1_new, W2_new, mask


# speedup vs baseline: 1.3697x; 1.2448x over previous
import jax
import jax.numpy as jnp
from jax.experimental import pallas as pl
from jax.experimental.pallas import tpu as pltpu

_DFF = 4096
_DMODEL = 1024
_K = 1024
_MB = 512
_NBLK = _DFF // _MB


def _bottom_k_mask(m):
    u = jax.lax.bitcast_convert_type(m, jnp.int32)
    k = jnp.int32(_K)

    def bs_body(_, carry):
        lo, hi = carry
        mid = lo + (hi - lo) // 2
        c = jnp.sum((u <= mid).astype(jnp.int32))
        take = c >= k
        return jnp.where(take, lo, mid + 1), jnp.where(take, mid, hi)

    _, p = jax.lax.fori_loop(
        0, 31, bs_body, (jnp.int32(0), jnp.int32(0x7F800000)))

    lt = u < p
    eq = u == p
    c_lt = jnp.sum(lt.astype(jnp.int32))
    need = k - c_lt

    e = eq.astype(jnp.int32)
    x = e
    s = 1
    while s < _MB:
        sh = jnp.concatenate([jnp.zeros((_NBLK, s), jnp.int32), x[:, :-s]],
                             axis=1)
        x = x + sh
        s *= 2
    row_tot = x[:, _MB - 1:_MB]
    y = row_tot
    s = 1
    while s < _NBLK:
        shy = jnp.concatenate([jnp.zeros((s, 1), jnp.int32), y[:-s, :]],
                              axis=0)
        y = y + shy
        s *= 2
    row_off = jnp.concatenate([jnp.zeros((1, 1), jnp.int32), y[:-1, :]],
                              axis=0)
    excl = (x - e) + row_off
    prune_eq = eq & (excl < need)
    keep = jnp.logical_not(jnp.logical_or(lt, prune_eq))
    return keep.astype(jnp.float32)


def _fused_body(w1_ref, w2_ref, f1_ref, f2_ref,
                maskout_ref, w1out_ref, w2out_ref,
                mags_s, mask_s):
    i = pl.program_id(0)

    @pl.when(i < _NBLK)
    def _mags_phase():
        w1 = w1_ref[pl.ds(i * _MB, _MB), :]
        w2 = w2_ref[:, pl.ds(i * _MB, _MB)]
        s1 = jnp.sum(w1 * w1, axis=1)
        s2 = jnp.sum(w2 * w2, axis=0)
        mags_s[pl.ds(i, 1), :] = (jnp.sqrt(s1) * jnp.sqrt(s2)).reshape(1, _MB)

    @pl.when(i == _NBLK)
    def _mask_phase():
        mask = _bottom_k_mask(mags_s[...])
        mask_s[...] = mask
        maskout_ref[...] = mask

    @pl.when(i >= _NBLK)
    def _blend_phase():
        j = i - _NBLK
        mrow = mask_s[pl.ds(j, 1), :]
        keep_r = mrow > 0.5
        w2blk = w2_ref[:, pl.ds(j * _MB, _MB)]
        w2out_ref[...] = jnp.where(keep_r, w2blk, f2_ref[...])

        ii = jax.lax.broadcasted_iota(jnp.int32, (_MB, _MB), 0)
        jj = jax.lax.broadcasted_iota(jnp.int32, (_MB, _MB), 1)
        m_b = jnp.broadcast_to(mrow, (_MB, _MB))
        mcol = jnp.sum(jnp.where(ii == jj, m_b, 0.0), axis=1, keepdims=True)
        keep_c = mcol > 0.5
        w1blk = w1_ref[pl.ds(j * _MB, _MB), :]
        w1out_ref[...] = jnp.where(keep_c, w1blk, f1_ref[...])


def _relu_body(x_ref, y_ref):
    y_ref[...] = jnp.maximum(x_ref[...], 0.0)


def kernel(x, W1, W2, frozen1, frozen2, target1, target2):
    del target1, target2

    mask2d, W1_new, W2_new = pl.pallas_call(
        _fused_body,
        grid=(2 * _NBLK,),
        in_specs=[
            pl.BlockSpec((_DFF, _DMODEL), lambda i: (0, 0)),
            pl.BlockSpec((_DMODEL, _DFF), lambda i: (0, 0)),
            pl.BlockSpec((_MB, _DMODEL),
                         lambda i: (jnp.maximum(i - _NBLK, 0), 0)),
            pl.BlockSpec((_DMODEL, _MB),
                         lambda i: (0, jnp.maximum(i - _NBLK, 0))),
        ],
        out_specs=[
            pl.BlockSpec((_NBLK, _MB), lambda i: (0, 0)),
            pl.BlockSpec((_MB, _DMODEL),
                         lambda i: (jnp.maximum(i - _NBLK, 0), 0)),
            pl.BlockSpec((_DMODEL, _MB),
                         lambda i: (0, jnp.maximum(i - _NBLK, 0))),
        ],
        out_shape=[
            jax.ShapeDtypeStruct((_NBLK, _MB), jnp.float32),
            jax.ShapeDtypeStruct((_DFF, _DMODEL), jnp.float32),
            jax.ShapeDtypeStruct((_DMODEL, _DFF), jnp.float32),
        ],
        scratch_shapes=[
            pltpu.VMEM((_NBLK, _MB), jnp.float32),
            pltpu.VMEM((_NBLK, _MB), jnp.float32),
        ],
    )(W1, W2, frozen1, frozen2)

    mask = mask2d.reshape(_DFF)

    x2 = x.reshape(-1, _DMODEL)
    xb = 1024
    y = pl.pallas_call(
        _relu_body,
        grid=(x2.shape[0] // xb,),
        in_specs=[pl.BlockSpec((xb, _DMODEL), lambda i: (i, 0))],
        out_specs=pl.BlockSpec((xb, _DMODEL), lambda i: (i, 0)),
        out_shape=jax.ShapeDtypeStruct(x2.shape, jnp.float32),
    )(x2)

    return y.reshape(x.shape), W1_new, W2_new, mask
